# Initial kernel scaffold; baseline (speedup 1.0000x reference)
import jax
import jax.numpy as jnp
from jax.experimental import pallas as pl


def kernel(x_nchw, patch_w2, patch_b, pos_tok, in_w, in_b, out_w, out_b, ln1_g, ln1_b, ff1_w, ff1_b, ff2_w, ff2_b, ln2_g, ln2_b, proj_w, proj_b, bn_g, bn_b, bn_mean, bn_var, mlp_w, mlp_b):
    raise NotImplementedError("write your pallas kernel here")


# speedup vs baseline: 6.0522x; 6.0522x over previous
import math

import numpy as np

import jax
import jax.numpy as jnp
from jax.experimental import pallas as pl
from jax.experimental.pallas import tpu as pltpu

S = 36
D = 32
HEADS = 4
DH = 8
FF = 2048
LAYERS = 2
NCLS = 36
PATCH = 2
CIN = 4
CPP = CIN * PATCH * PATCH
LN_EPS = 1e-5
BN_EPS = 1e-5

E_GRP = 4
R = E_GRP * S
RQ = HEADS * R
BLK = 128
FF_CHUNK = 512
HBLK = 512

_SCALE = 1.0 / math.sqrt(DH)
_INV_SQRT2 = 1.0 / math.sqrt(2.0)
_NEG = -1e30


def _ln(z, g, b):
    mu = jnp.mean(z, axis=-1, keepdims=True)
    d = z - mu
    var = jnp.mean(d * d, axis=-1, keepdims=True)
    return d * jax.lax.rsqrt(var + LN_EPS) * g + b


def _erf_poly(x):
    p = 0.3275911
    a1, a2, a3, a4, a5 = (0.254829592, -0.284496736, 1.421413741,
                          -1.453152027, 1.061405429)
    sgn = jnp.where(x < 0.0, -1.0, 1.0)
    ax = jnp.abs(x)
    t = 1.0 / (1.0 + p * ax)
    poly = ((((a5 * t + a4) * t + a3) * t + a2) * t + a1) * t
    return sgn * (1.0 - poly * jnp.exp(-ax * ax))


def _encoder_kernel(pat_ref, pw_ref, posb_ref, inw_ref, inb_ref,
                    wexp_ref, outb_ref, ln1g_ref, ln1b_ref,
                    ff1w_ref, ff1b_ref, ff2w_ref, ff2b_ref,
                    ln2g_ref, ln2b_ref, hm_ref, va_ref, fm_ref, am_ref,
                    h_ref):
    M = BLK * S
    x = jnp.dot(pat_ref[...], pw_ref[...],
                preferred_element_type=jnp.float32) + posb_ref[...]
    h = x

    for l in range(LAYERS):
        hb = h.astype(jnp.bfloat16)
        qkv = jnp.dot(hb, inw_ref[l],
                      preferred_element_type=jnp.float32) + inb_ref[l]
        q = qkv[:, :D]
        kb = qkv[:, D:2 * D].astype(jnp.bfloat16)
        v = qkv[:, 2 * D:]
        parts = []
        for g in range(BLK // E_GRP):
            sl = slice(g * R, (g + 1) * R)
            qg, kg, vg = q[sl], kb[sl], v[sl]
            qstack = jnp.concatenate(
                [qg * hm_ref[hh:hh + 1, :] for hh in range(HEADS)],
                axis=0).astype(jnp.bfloat16)
            s = jax.lax.dot_general(
                qstack, kg, (((1,), (1,)), ((), ())),
                preferred_element_type=jnp.float32)
            e = jnp.exp(s + am_ref[...]).astype(jnp.bfloat16)
            vaug = jnp.concatenate(
                [vg * hm_ref[hh:hh + 1, :] + va_ref[hh:hh + 1, :]
                 for hh in range(HEADS)],
                axis=1).astype(jnp.bfloat16)
            u = jnp.dot(e, vaug,
                        preferred_element_type=jnp.float32)
            acc = None
            for hh in range(HEADS):
                ub = u[hh * R:(hh + 1) * R]
                ch = 32 * hh + (8 * hh + 8) % 32
                r = pl.reciprocal(ub[:, ch:ch + 1], approx=True)
                part = ub * fm_ref[hh:hh + 1, :] * r
                acc = part if acc is None else acc + part
            parts.append(acc)
        sfull = jnp.concatenate(parts, axis=0)
        attn = jnp.dot(sfull.astype(jnp.bfloat16), wexp_ref[l],
                       preferred_element_type=jnp.float32) + outb_ref[l]
        h = _ln(h + attn, ln1g_ref[l], ln1b_ref[l])

        hb2 = h.astype(jnp.bfloat16)
        acc = jnp.zeros((M, D), jnp.float32)
        for c in range(0, FF, FF_CHUNK):
            t = jnp.dot(hb2, ff1w_ref[l, :, c:c + FF_CHUNK],
                        preferred_element_type=jnp.float32)
            t = jnp.maximum(t + ff1b_ref[l, :, c:c + FF_CHUNK],
                            0.0).astype(jnp.bfloat16)
            acc = acc + jnp.dot(t, ff2w_ref[l, c:c + FF_CHUNK, :],
                                preferred_element_type=jnp.float32)
        h = _ln(h + acc + ff2b_ref[l], ln2g_ref[l], ln2b_ref[l])

    h_ref[...] = h


def _head_kernel(f_ref, pw_ref, pb_ref, sc_ref, sh_ref, mw_ref, mb_ref,
                 o_ref):
    y = jnp.dot(f_ref[...], pw_ref[...],
                preferred_element_type=jnp.float32) + pb_ref[...]
    y = y * sc_ref[...] + sh_ref[...]
    y = 0.5 * y * (1.0 + _erf_poly(y * _INV_SQRT2))
    z = jnp.dot(y.astype(jnp.bfloat16), mw_ref[...],
                preferred_element_type=jnp.float32) + mb_ref[...]
    o_ref[...] = jnp.maximum(z, 0.0)


def _const(x):
    return jnp.asarray(x)


def kernel(x_nchw, patch_w2, patch_b, pos_tok, in_w, in_b, out_w, out_b,
           ln1_g, ln1_b, ff1_w, ff1_b, ff2_w, ff2_b, ln2_g, ln2_b,
           proj_w, proj_b, bn_g, bn_b, bn_mean, bn_var, mlp_w, mlp_b):
    B = x_nchw.shape[0]
    bf = jnp.bfloat16

    Hp = x_nchw.shape[2] // PATCH
    Wp = x_nchw.shape[3] // PATCH
    patches = x_nchw.reshape(B, CIN, Hp, PATCH, Wp, PATCH)
    patches = patches.transpose(0, 2, 4, 1, 3, 5).reshape(B * S, CPP)
    patches = patches.astype(bf)

    posb = jnp.tile(pos_tok + patch_b, (BLK, 1))

    qscale = jnp.concatenate(
        [jnp.full((1, D), _SCALE, jnp.float32),
         jnp.ones((1, 2 * D), jnp.float32)], axis=1)
    in_w_s = (in_w * qscale[None]).astype(bf)
    in_b_s = in_b * qscale[None]

    hm = np.zeros((HEADS, D), np.float32)
    va = np.zeros((HEADS, D), np.float32)
    fm = np.zeros((HEADS, HEADS * D), np.float32)
    pmat = np.zeros((HEADS * D, D), np.float32)
    for h in range(HEADS):
        hm[h, DH * h:DH * (h + 1)] = 1.0
        va[h, (DH * h + DH) % D] = 1.0
        fm[h, D * h + DH * h:D * h + DH * (h + 1)] = 1.0
        for d in range(DH):
            pmat[D * h + DH * h + d, DH * h + d] = 1.0
    wexp = jnp.einsum("tr,lrd->ltd", _const(pmat), out_w)

    ri = np.arange(R)[:, None] // S
    ci = np.arange(R)[None, :] // S
    am = np.where(ri == ci, 0.0, _NEG).astype(np.float32)
    am = np.tile(am, (HEADS, 1))

    nsteps = B // BLK
    c0 = lambda i: (0, 0)
    c03 = lambda i: (0, 0, 0)
    enc_specs = [
        pl.BlockSpec((BLK * S, CPP), lambda i: (i, 0)),
        pl.BlockSpec((CPP, D), c0),
        pl.BlockSpec((BLK * S, D), c0),
        pl.BlockSpec((LAYERS, D, 3 * D), c03),
        pl.BlockSpec((LAYERS, 1, 3 * D), c03),
        pl.BlockSpec((LAYERS, HEADS * D, D), c03),
        pl.BlockSpec((LAYERS, 1, D), c03),
        pl.BlockSpec((LAYERS, 1, D), c03),
        pl.BlockSpec((LAYERS, 1, D), c03),
        pl.BlockSpec((LAYERS, D, FF), c03),
        pl.BlockSpec((LAYERS, 1, FF), c03),
        pl.BlockSpec((LAYERS, FF, D), c03),
        pl.BlockSpec((LAYERS, 1, D), c03),
        pl.BlockSpec((LAYERS, 1, D), c03),
        pl.BlockSpec((LAYERS, 1, D), c03),
        pl.BlockSpec((HEADS, D), c0),
        pl.BlockSpec((HEADS, D), c0),
        pl.BlockSpec((HEADS, HEADS * D), c0),
        pl.BlockSpec((RQ, R), c0),
    ]
    h_out = pl.pallas_call(
        _encoder_kernel,
        out_shape=jax.ShapeDtypeStruct((B * S, D), jnp.float32),
        grid=(nsteps,),
        in_specs=enc_specs,
        out_specs=pl.BlockSpec((BLK * S, D), lambda i: (i, 0)),
        compiler_params=pltpu.CompilerParams(
            dimension_semantics=("parallel",)),
    )(patches, patch_w2.astype(bf), posb, in_w_s, in_b_s,
      wexp.astype(bf), out_b, ln1_g, ln1_b,
      ff1_w.astype(bf), ff1_b, ff2_w.astype(bf), ff2_b, ln2_g, ln2_b,
      _const(hm), _const(va), _const(fm), _const(am))

    flat = h_out.reshape(B, S * D).astype(bf)
    bn_scale = bn_g * jax.lax.rsqrt(bn_var + BN_EPS)
    bn_shift = bn_b - bn_mean * bn_scale
    head_specs = [
        pl.BlockSpec((HBLK, S * D), lambda i: (i, 0)),
        pl.BlockSpec((S * D, 2 * D), c0),
        pl.BlockSpec((1, 2 * D), c0),
        pl.BlockSpec((1, 2 * D), c0),
        pl.BlockSpec((1, 2 * D), c0),
        pl.BlockSpec((2 * D, NCLS), c0),
        pl.BlockSpec((1, NCLS), c0),
    ]
    return pl.pallas_call(
        _head_kernel,
        out_shape=jax.ShapeDtypeStruct((B, NCLS), jnp.float32),
        grid=(B // HBLK,),
        in_specs=head_specs,
        out_specs=pl.BlockSpec((HBLK, NCLS), lambda i: (i, 0)),
        compiler_params=pltpu.CompilerParams(
            dimension_semantics=("parallel",)),
    )(flat, proj_w.astype(bf), proj_b, bn_scale, bn_shift,
      mlp_w.astype(bf), mlp_b)
